# baseline (speedup 1.0000x reference)
import functools

import jax
import jax.numpy as jnp
from jax import lax
from jax.experimental import pallas as pl
from jax.experimental.pallas import tpu as pltpu
from jax.experimental.pallas import tpu_sc as plsc

CHUNK = 512
LANES = 16


@functools.lru_cache(maxsize=None)
def _build_gather(V, D, J, S):
    info = plsc.get_sparse_core_info()
    NC, NS = info.num_cores, info.num_subcores
    NW = NC * NS
    B = J * S
    cpj = S // CHUNK
    n_chunks = B // CHUNK
    assert n_chunks % NW == 0
    cpw = n_chunks // NW
    assert cpj & (cpj - 1) == 0
    cpj_shift = cpj.bit_length() - 1
    mesh = plsc.VectorSubcoreMesh(core_axis_name="c", subcore_axis_name="s")

    @functools.partial(
        pl.kernel,
        mesh=mesh,
        out_type=jax.ShapeDtypeStruct((J, D, S), jnp.float32),
        scratch_types=[
            pltpu.VMEM((CHUNK,), jnp.int32),
            pltpu.VMEM((CHUNK, D), jnp.float32),
            pltpu.VMEM((D, CHUNK), jnp.float32),
            pltpu.SemaphoreType.DMA,
        ],
        compiler_params=pltpu.CompilerParams(
            use_tc_tiling_on_sc=False, needs_layout_passes=False),
    )
    def gather_k(table_hbm, idx_hbm, out_hbm, idx_v, gbuf, tbuf, sem):
        wid = lax.axis_index("s") * NC + lax.axis_index("c")
        c0 = wid * cpw
        i16 = lax.iota(jnp.int32, LANES)

        def step(c, carry):
            j = c >> cpj_shift
            s0 = (c & (cpj - 1)) * CHUNK
            pltpu.sync_copy(idx_hbm.at[pl.ds(c * CHUNK, CHUNK)], idx_v)
            pltpu.async_copy(table_hbm.at[idx_v], gbuf, sem).wait()

            def trans_row(d, carry2):
                dcol = jnp.full((LANES,), d, dtype=jnp.int32)
                for g in range(CHUNK // LANES):
                    vals = plsc.load_gather(gbuf, [i16 + (g * LANES), dcol])
                    tbuf[d, pl.ds(g * LANES, LANES)] = vals
                return carry2

            lax.fori_loop(0, D, trans_row, 0)
            pltpu.sync_copy(tbuf, out_hbm.at[j, :, pl.ds(s0, CHUNK)])
            return carry

        lax.fori_loop(c0, c0 + cpw, step, 0)

    return gather_k


def kernel(input, weight):
    B0, B1 = input.shape
    V, D = weight.shape
    idx = input.T.reshape(-1).astype(jnp.int32)
    out = _build_gather(V, D, B1, B0)(weight, idx)
    return jnp.transpose(out, (2, 0, 1))


# speedup vs baseline: 1.7301x; 1.7301x over previous
import functools

import jax
import jax.numpy as jnp
from jax import lax
from jax.experimental import pallas as pl
from jax.experimental.pallas import tpu as pltpu
from jax.experimental.pallas import tpu_sc as plsc

CHUNK = 256
LANES = 16
GRP = CHUNK // LANES


@functools.lru_cache(maxsize=None)
def _build_gather(V, D, J, S):
    info = plsc.get_sparse_core_info()
    NC, NS = info.num_cores, info.num_subcores
    NW = NC * NS
    B = J * S
    D2 = 2 * D
    cpj = S // CHUNK
    n_chunks = B // CHUNK
    assert n_chunks % (2 * NW) == 0
    cpw = n_chunks // NW
    assert cpj & (cpj - 1) == 0
    cpj_shift = cpj.bit_length() - 1
    PITCH = CHUNK + 1
    mesh = plsc.VectorSubcoreMesh(core_axis_name="c", subcore_axis_name="s")

    @functools.partial(
        pl.kernel,
        mesh=mesh,
        out_type=jax.ShapeDtypeStruct((J, D, S), jnp.float32),
        scratch_types=[
            pltpu.VMEM((CHUNK,), jnp.int32),
            [pltpu.VMEM((CHUNK,), jnp.int32) for _ in range(2)],
            [pltpu.VMEM((CHUNK,), jnp.int32) for _ in range(2)],
            [pltpu.VMEM((CHUNK, D2), jnp.float32) for _ in range(2)],
            [pltpu.VMEM((D, PITCH), jnp.float32) for _ in range(2)],
            [pltpu.SemaphoreType.DMA for _ in range(2)],
            [pltpu.SemaphoreType.DMA for _ in range(2)],
        ],
        compiler_params=pltpu.CompilerParams(
            use_tc_tiling_on_sc=False, needs_layout_passes=False),
    )
    def gather_k(table_hbm, idx_hbm, out_hbm, tmp_v, idx2_v, off_v,
                 gbufs, tbufs, gsems, ssems):
        wid = lax.axis_index("s") * NC + lax.axis_index("c")
        c0 = wid * cpw
        i16 = lax.iota(jnp.int32, LANES)

        def load_indices(c, b):
            pltpu.sync_copy(idx_hbm.at[pl.ds(c * CHUNK, CHUNK)], tmp_v)
            for g in range(GRP):
                v = tmp_v[pl.ds(g * LANES, LANES)]
                idx2_v[b][pl.ds(g * LANES, LANES)] = v >> 1
                off_v[b][pl.ds(g * LANES, LANES)] = (v & 1) * D

        def fire_gather(b):
            pltpu.make_async_copy(
                table_hbm.at[idx2_v[b]], gbufs[b], gsems[b]).start()

        def wait_gather(b):
            pltpu.make_async_copy(
                table_hbm.at[idx2_v[b]], gbufs[b], gsems[b]).wait()

        def store_cp(c, b):
            j = c >> cpj_shift
            s0 = (c & (cpj - 1)) * CHUNK
            return pltpu.make_async_copy(
                tbufs[b].at[:, pl.ds(0, CHUNK)],
                out_hbm.at[j, :, pl.ds(s0, CHUNK)], ssems[b])

        def transpose(b):
            gbuf = gbufs[b]
            tbuf = tbufs[b]
            offs = off_v[b]

            def sgroup(g, carry):
                og = offs[pl.ds(g * LANES, LANES)]
                for i in range(LANES):
                    s = g * LANES + i
                    offb = lax.gather(
                        og, jnp.full((LANES, 1), i, jnp.int32),
                        lax.GatherDimensionNumbers(
                            offset_dims=(), collapsed_slice_dims=(0,),
                            start_index_map=(0,)),
                        (1,), mode=lax.GatherScatterMode.PROMISE_IN_BOUNDS)
                    srow = jnp.full((LANES,), 0, jnp.int32) + s
                    for k in range(D // LANES):
                        colv = offb + (i16 + k * LANES)
                        vals = plsc.load_gather(gbuf, [srow, colv])
                        plsc.store_scatter(
                            tbuf, [i16 + k * LANES, srow], vals)
                return carry

            lax.fori_loop(0, GRP, sgroup, 0)

        load_indices(c0, 0)
        fire_gather(0)
        load_indices(c0 + 1, 1)
        fire_gather(1)

        def group(t, carry):
            for b in range(2):
                i = 2 * t + b
                c = c0 + i
                wait_gather(b)

                @pl.when(i >= 2)
                def _():
                    store_cp(c - 2, b).wait()

                transpose(b)
                store_cp(c, b).start()

                @pl.when(i + 2 < cpw)
                def _():
                    load_indices(c + 2, b)
                    fire_gather(b)

            return carry

        lax.fori_loop(0, cpw // 2, group, 0)
        store_cp(c0 + cpw - 2, 0).wait()
        store_cp(c0 + cpw - 1, 1).wait()

    return gather_k


def kernel(input, weight):
    B0, B1 = input.shape
    V, D = weight.shape
    idx = input.T.reshape(-1).astype(jnp.int32)
    table2 = weight.reshape(V // 2, 2 * D)
    out = _build_gather(V, D, B1, B0)(table2, idx)
    return jnp.transpose(out, (2, 0, 1))
